# baseline (speedup 1.0000x reference)
import jax
import jax.numpy as jnp
from jax.experimental import pallas as pl
from jax.experimental.pallas import tpu as pltpu

_G = 4
_NPACK = 25
_TB = 2048


def _body(x_ref, w_ref, b_ref, o_ref):
    G, I, B = x_ref.shape
    strips = w_ref[0]
    wbd = jnp.concatenate(
        [
            jnp.pad(strips[32 * g:32 * (g + 1), :], ((0, 0), (I * g, I * (G - 1 - g))))
            for g in range(G)
        ],
        axis=0,
    )
    bias = b_ref[0]
    for t in range(B // _TB):
        sl = slice(t * _TB, (t + 1) * _TB)
        xb = x_ref[:, :, sl].reshape(G * I, _TB).astype(jnp.bfloat16)
        acc = jnp.dot(wbd, xb, preferred_element_type=jnp.float32)
        o_ref[:, :, sl] = acc.reshape(G, 32, _TB) + bias


@jax.jit
def _attribute_embed(x, W, b):
    B, N, I = x.shape
    O = W.shape[2]
    npad = _NPACK * _G - N

    xt = x.transpose(1, 2, 0)

    Wpad = jnp.concatenate([W, jnp.zeros((npad, I, O), W.dtype)], axis=0)
    Wstrip = Wpad.transpose(0, 2, 1).reshape(_NPACK, _G * O, I).astype(jnp.bfloat16)
    b8 = jnp.concatenate(
        [b, jnp.zeros((npad, O), b.dtype)], axis=0
    ).reshape(_NPACK, _G, O, 1)

    y = pl.pallas_call(
        _body,
        grid=(_NPACK,),
        in_specs=[
            pl.BlockSpec((_G, I, B), lambda p: (p, 0, 0)),
            pl.BlockSpec((1, _G * O, I), lambda p: (p, 0, 0)),
            pl.BlockSpec((1, _G, O, 1), lambda p: (p, 0, 0, 0)),
        ],
        out_specs=pl.BlockSpec((_G, O, B), lambda p: (p, 0, 0)),
        out_shape=jax.ShapeDtypeStruct((N, O, B), jnp.float32),
        compiler_params=pltpu.CompilerParams(
            dimension_semantics=("parallel",),
        ),
    )(xt, Wstrip, b8)

    return y.transpose(2, 0, 1)


def kernel(x, W, b):
    return _attribute_embed(x, W, b)


# speedup vs baseline: 1.0402x; 1.0402x over previous
import jax
import jax.numpy as jnp
from jax.experimental import pallas as pl
from jax.experimental.pallas import tpu as pltpu

_G = 8
_NPACK = 13
_TB = 4096


def _body(x_ref, w_ref, b_ref, o_ref):
    G, I, B = x_ref.shape
    strips = w_ref[0]
    wbd = jnp.concatenate(
        [
            jnp.pad(strips[32 * g:32 * (g + 1), :], ((0, 0), (I * g, I * (G - 1 - g))))
            for g in range(G)
        ],
        axis=0,
    )
    bias = b_ref[0]
    for t in range(B // _TB):
        sl = slice(t * _TB, (t + 1) * _TB)
        xb = x_ref[:, :, sl].reshape(G * I, _TB).astype(jnp.bfloat16)
        acc = jnp.dot(wbd, xb, preferred_element_type=jnp.float32)
        o_ref[:, :, sl] = acc.reshape(G, 32, _TB) + bias


@jax.jit
def _attribute_embed(x, W, b):
    B, N, I = x.shape
    O = W.shape[2]
    npad = _NPACK * _G - N

    xt = x.transpose(1, 2, 0)

    Wpad = jnp.concatenate([W, jnp.zeros((npad, I, O), W.dtype)], axis=0)
    Wstrip = Wpad.transpose(0, 2, 1).reshape(_NPACK, _G * O, I).astype(jnp.bfloat16)
    b8 = jnp.concatenate(
        [b, jnp.zeros((npad, O), b.dtype)], axis=0
    ).reshape(_NPACK, _G, O, 1)

    y = pl.pallas_call(
        _body,
        grid=(_NPACK,),
        in_specs=[
            pl.BlockSpec((_G, I, B), lambda p: (p, 0, 0)),
            pl.BlockSpec((1, _G * O, I), lambda p: (p, 0, 0)),
            pl.BlockSpec((1, _G, O, 1), lambda p: (p, 0, 0, 0)),
        ],
        out_specs=pl.BlockSpec((_G, O, B), lambda p: (p, 0, 0)),
        out_shape=jax.ShapeDtypeStruct((N, O, B), jnp.float32),
        compiler_params=pltpu.CompilerParams(
            dimension_semantics=("parallel",),
        ),
    )(xt, Wstrip, b8)

    return y.transpose(2, 0, 1)


def kernel(x, W, b):
    return _attribute_embed(x, W, b)
